# Initial kernel scaffold; baseline (speedup 1.0000x reference)
import jax
import jax.numpy as jnp
from jax.experimental import pallas as pl


def kernel(logits):
    raise NotImplementedError("write your pallas kernel here")


# speedup vs baseline: 36.9722x; 36.9722x over previous
import sys

import jax
import jax.numpy as jnp
from jax.experimental import pallas as pl

_EPS = sys.float_info.epsilon
_V = 32768
_K = 3
_L = 16
_B = 8
_ROWS_PER_BLOCK = 8
_BIG_I32 = 2**30


def _stats_body(x_ref, lp_ref, ix_ref):
    x = x_ref[...]
    m = jnp.max(x, axis=1, keepdims=True)
    e = jnp.exp(x - m)
    z = jnp.sum(e, axis=1, keepdims=True)
    lp = jnp.log(e / z + _EPS)
    iota = jax.lax.broadcasted_iota(jnp.int32, lp.shape, 1)
    vals, idxs = [], []
    cur = lp
    for _ in range(_K):
        mj = jnp.max(cur, axis=1, keepdims=True)
        ij = jnp.min(jnp.where(cur == mj, iota, _BIG_I32), axis=1, keepdims=True)
        vals.append(mj)
        idxs.append(ij)
        cur = jnp.where(iota == ij, -jnp.inf, cur)
    pad_f = jnp.zeros((_ROWS_PER_BLOCK, 8 - _K), jnp.float32)
    pad_i = jnp.zeros((_ROWS_PER_BLOCK, 8 - _K), jnp.int32)
    lp_ref[...] = jnp.concatenate(vals + [pad_f], axis=1)
    ix_ref[...] = jnp.concatenate(idxs + [pad_i], axis=1)


def _beam_body(lp_ref, ix_ref, seq_ref, sc_ref):
    scores = lp_ref[0:_B, 0:_K]
    tok0 = ix_ref[0:_B, 0:_K]
    col = jax.lax.broadcasted_iota(jnp.int32, (_B, _L), 1)
    blocks = [jnp.where(col == 0, tok0[:, k : k + 1], 0) for k in range(_K)]
    for t in range(1, _L):
        lp = lp_ref[t * _B : (t + 1) * _B, 0:_K]
        ix = ix_ref[t * _B : (t + 1) * _B, 0:_K]
        cand = jnp.concatenate(
            [scores[:, k : k + 1] + lp for k in range(_K)], axis=1
        )
        flat = jnp.concatenate([ix + k * _V for k in range(_K)], axis=1)
        ss, ff = [], []
        for _ in range(_K):
            mj = jnp.max(cand, axis=1, keepdims=True)
            fj = jnp.min(jnp.where(cand == mj, flat, _BIG_I32), axis=1, keepdims=True)
            ss.append(mj)
            ff.append(fj)
            cand = jnp.where(flat == fj, -jnp.inf, cand)
        scores = jnp.concatenate(ss, axis=1)
        sel = jnp.concatenate(ff, axis=1)
        parent = sel >> 15
        token = sel & (_V - 1)
        nb = []
        for k in range(_K):
            pk = parent[:, k : k + 1]
            blk = jnp.where(pk == 0, blocks[0], jnp.where(pk == 1, blocks[1], blocks[2]))
            blk = jnp.where(col == t, token[:, k : k + 1], blk)
            nb.append(blk)
        blocks = nb
    seq_ref[...] = jnp.concatenate(blocks, axis=1)
    sc_ref[...] = jnp.concatenate(
        [scores, jnp.zeros((_B, 8 - _K), jnp.float32)], axis=1
    )


def kernel(logits):
    rows = logits.reshape(_B * _L, _V)
    n_blocks = (_B * _L) // _ROWS_PER_BLOCK
    lp3, ix3 = pl.pallas_call(
        _stats_body,
        grid=(n_blocks,),
        in_specs=[pl.BlockSpec((_ROWS_PER_BLOCK, _V), lambda i: (i, 0))],
        out_specs=[
            pl.BlockSpec((_ROWS_PER_BLOCK, 8), lambda i: (i, 0)),
            pl.BlockSpec((_ROWS_PER_BLOCK, 8), lambda i: (i, 0)),
        ],
        out_shape=[
            jax.ShapeDtypeStruct((_B * _L, 8), jnp.float32),
            jax.ShapeDtypeStruct((_B * _L, 8), jnp.int32),
        ],
    )(rows)
    lp_t = lp3.reshape(_B, _L, 8).transpose(1, 0, 2).reshape(_B * _L, 8)
    ix_t = ix3.reshape(_B, _L, 8).transpose(1, 0, 2).reshape(_B * _L, 8)
    seq, sc = pl.pallas_call(
        _beam_body,
        out_shape=[
            jax.ShapeDtypeStruct((_B, _K * _L), jnp.int32),
            jax.ShapeDtypeStruct((_B, 8), jnp.float32),
        ],
    )(lp_t, ix_t)
    tokens = seq.reshape(_B, _K, _L).transpose(0, 2, 1)
    return tokens, sc[:, :_K]
